# baseline (speedup 1.0000x reference)
import jax
import jax.numpy as jnp
from jax import lax
from jax.experimental import pallas as pl
from jax.experimental.pallas import tpu as pltpu
from jax.experimental.pallas import tpu_sc as plsc

N_FIELDS = 26
VOCAB = 100000
DIM = 32
BATCH = 4096
NC, NS = 2, 16
NW = NC * NS
ROWS = BATCH // NW
L = 16
CHUNKS = (13, 13)


def _make_body(nf):
    def _body(tf, idx_t, *refs):
        outs = refs[:nf]
        idx_v = refs[nf]
        tbufs = refs[nf + 1:nf + 3]
        gsems = refs[nf + 3:nf + 5]
        wsems = refs[nf + 5:nf + 7]
        wid = lax.axis_index("s") * NC + lax.axis_index("c")
        base = wid * ROWS
        pltpu.sync_copy(idx_t.at[:, pl.ds(base, ROWS)], idx_v)
        coff_lo = lax.broadcasted_iota(jnp.int32, (L,), 0) * VOCAB
        coff_hi = coff_lo + L * VOCAB

        def gather(j):
            t = tbufs[j % 2]

            def chunk(k, _):
                v = idx_v[j, pl.ds(k * L, L)]
                for l in range(L):
                    r = v[l]
                    pltpu.async_copy(tf.at[j].at[coff_lo + r],
                                     t.at[k * L + l, pl.ds(0, L)],
                                     gsems[j % 2])
                    pltpu.async_copy(tf.at[j].at[coff_hi + r],
                                     t.at[k * L + l, pl.ds(L, L)],
                                     gsems[j % 2])
                return _

            lax.fori_loop(0, ROWS // L, chunk, 0)

        def drain_gather(j):
            pltpu.make_async_copy(outs[j].at[pl.ds(base, ROWS)],
                                  tbufs[j % 2], gsems[j % 2]).wait()

        wd = [None] * nf
        gather(0)
        for j in range(nf):
            drain_gather(j)
            if j + 1 < nf:
                gather(j + 1)
            if j >= 2:
                wd[j - 2].wait()
            wd[j] = pltpu.async_copy(tbufs[j % 2],
                                     outs[j].at[pl.ds(base, ROWS)],
                                     wsems[j % 2])
        if nf >= 2:
            wd[nf - 2].wait()
        wd[nf - 1].wait()

    return _body


def kernel(inputs, tables):
    idx_t = inputs.T
    mesh = plsc.VectorSubcoreMesh(
        core_axis_name="c", subcore_axis_name="s",
        num_cores=NC, num_subcores=NS)
    outs = []
    f0 = 0
    for nf in CHUNKS:
        tf = jnp.transpose(tables[f0:f0 + nf], (0, 2, 1))
        tf = tf.reshape(nf, DIM * VOCAB)
        k = pl.kernel(
            _make_body(nf),
            out_type=[jax.ShapeDtypeStruct((BATCH, DIM), jnp.float32)] * nf,
            mesh=mesh,
            scratch_types=(
                [pltpu.VMEM((nf, ROWS), jnp.int32)]
                + [pltpu.VMEM((ROWS, DIM), jnp.float32)] * 2
                + [pltpu.SemaphoreType.DMA] * 4
            ),
            compiler_params=pltpu.CompilerParams(use_tc_tiling_on_sc=False,
                                                 needs_layout_passes=False),
        )
        outs.extend(k(tf, idx_t[f0:f0 + nf]))
        f0 += nf
    return tuple(o.reshape(-1) for o in outs)


# speedup vs baseline: 1.2288x; 1.2288x over previous
import jax
import jax.numpy as jnp
from jax import lax
from jax.experimental import pallas as pl
from jax.experimental.pallas import tpu as pltpu
from jax.experimental.pallas import tpu_sc as plsc

N_FIELDS = 26
VOCAB = 100000
DIM = 32
BATCH = 4096
NC, NS = 2, 16
NW = NC * NS
ROWS = BATCH // NW
L = 16


def _body(tf, idx_t, *refs):
    outs = refs[:N_FIELDS]
    idx_v = refs[N_FIELDS]
    tbufs = refs[N_FIELDS + 1:N_FIELDS + 3]
    gsems = refs[N_FIELDS + 3:N_FIELDS + 5]
    wsems = refs[N_FIELDS + 5:N_FIELDS + 7]
    wid = lax.axis_index("s") * NC + lax.axis_index("c")
    base = wid * ROWS
    pltpu.sync_copy(idx_t.at[:, pl.ds(base, ROWS)], idx_v)
    coff_lo = lax.broadcasted_iota(jnp.int32, (L,), 0) * VOCAB
    coff_hi = coff_lo + L * VOCAB

    def gather(j):
        t = tbufs[j % 2]

        def chunk(k, _):
            v = idx_v[j, pl.ds(k * L, L)]
            for l in range(L):
                r = v[l]
                pltpu.async_copy(tf.at[j].at[coff_lo + r],
                                 t.at[k * L + l, pl.ds(0, L)], gsems[j % 2])
                pltpu.async_copy(tf.at[j].at[coff_hi + r],
                                 t.at[k * L + l, pl.ds(L, L)], gsems[j % 2])
            return _

        lax.fori_loop(0, ROWS // L, chunk, 0)

    def drain_gather(j):
        pltpu.make_async_copy(outs[j].at[pl.ds(base, ROWS)],
                              tbufs[j % 2], gsems[j % 2]).wait()

    wd = [None] * N_FIELDS
    gather(0)
    for j in range(N_FIELDS):
        drain_gather(j)
        if j + 1 < N_FIELDS:
            gather(j + 1)
        if j >= 2:
            wd[j - 2].wait()
        wd[j] = pltpu.async_copy(tbufs[j % 2],
                                 outs[j].at[pl.ds(base, ROWS)], wsems[j % 2])
    wd[N_FIELDS - 2].wait()
    wd[N_FIELDS - 1].wait()


def kernel(inputs, tables):
    idx_t = inputs.T
    tf = jnp.transpose(tables, (0, 2, 1)).reshape(N_FIELDS, DIM * VOCAB)
    mesh = plsc.VectorSubcoreMesh(
        core_axis_name="c", subcore_axis_name="s",
        num_cores=NC, num_subcores=NS)
    k = pl.kernel(
        _body,
        out_type=[jax.ShapeDtypeStruct((BATCH, DIM), jnp.float32)] * N_FIELDS,
        mesh=mesh,
        scratch_types=(
            [pltpu.VMEM((N_FIELDS, ROWS), jnp.int32)]
            + [pltpu.VMEM((ROWS, DIM), jnp.float32)] * 2
            + [pltpu.SemaphoreType.DMA] * 4
        ),
        compiler_params=pltpu.CompilerParams(use_tc_tiling_on_sc=False,
                                             needs_layout_passes=False),
    )
    outs = k(tf, idx_t)
    return tuple(o.reshape(-1) for o in outs)
